# baseline (speedup 1.0000x reference)
import jax
import jax.numpy as jnp
from jax import lax
from jax.experimental import pallas as pl
from jax.experimental.pallas import tpu as pltpu
from jax.experimental.pallas import tpu_sc as plsc

N = 10000
E = 160000
D = 256
DE = 16
H = 128
NC = 2
NS = 16
L = 16
RB = 128
CH = 10752
EP = CH * NS
NB = CH // RB
PAD = EP - E - N
NROW = N // NS
HQ = 64
NQ = 4
NP = 10240
NRD = NP // NS
GB = 5
BN = N // GB
EB = 2000

_f32 = jnp.float32


def _wg(w_ref, avs_ref, avd_ref):
    w = w_ref[...]
    cs = jnp.sum(w * avs_ref[...][None, :], axis=1)
    cd = jnp.sum(w * avd_ref[...][None, :], axis=1)
    return jnp.concatenate(
        [w, cs[:, None], cd[:, None], jnp.zeros((D, H - 2), _f32)], axis=1)


def _split_out(xl, oq_ref, oa_ref):
    for q in range(NQ):
        oq_ref[q] = xl[:, q * HQ:(q + 1) * HQ].astype(jnp.bfloat16)
    oa_ref[...] = xl[:, D:]


def _node_body(x_ref, w_ref, avs_ref, avd_ref, oq_ref, oa_ref):
    xl = jnp.dot(x_ref[...], _wg(w_ref, avs_ref, avd_ref),
                 preferred_element_type=_f32)
    _split_out(xl, oq_ref, oa_ref)


_node = pl.pallas_call(
    _node_body,
    grid=(GB,),
    in_specs=[
        pl.BlockSpec((BN, D), lambda i: (i, 0)),
        pl.BlockSpec((D, D), lambda i: (0, 0)),
        pl.BlockSpec((D,), lambda i: (0,)),
        pl.BlockSpec((D,), lambda i: (0,)),
    ],
    out_specs=[
        pl.BlockSpec((NQ, BN, HQ), lambda i: (0, i, 0)),
        pl.BlockSpec((BN, H), lambda i: (i, 0)),
    ],
    out_shape=[
        jax.ShapeDtypeStruct((NQ, N, HQ), jnp.bfloat16),
        jax.ShapeDtypeStruct((N, H), _f32),
    ],
)


def _ln(h, g_ref, be_ref):
    m = jnp.mean(h, axis=1, keepdims=True)
    v = jnp.mean((h - m) ** 2, axis=1, keepdims=True)
    return (h - m) / jnp.sqrt(v + 1e-5) * g_ref[...][None, :] + be_ref[...][None, :]


def _mid_body(o_ref, b_ref, g_ref, be_ref, w_ref, avs_ref, avd_ref,
              oq_ref, oa_ref):
    o = jnp.concatenate([o_ref[q] for q in range(NQ)], axis=1)
    h = jnp.maximum(o + b_ref[...][None, :], 0.0)
    h = _ln(h, g_ref, be_ref)
    xl = jnp.dot(h, _wg(w_ref, avs_ref, avd_ref), preferred_element_type=_f32)
    _split_out(xl, oq_ref, oa_ref)


_mid = pl.pallas_call(
    _mid_body,
    grid=(GB,),
    in_specs=[
        pl.BlockSpec((NQ, BN, HQ), lambda i: (0, i, 0)),
        pl.BlockSpec((D,), lambda i: (0,)),
        pl.BlockSpec((D,), lambda i: (0,)),
        pl.BlockSpec((D,), lambda i: (0,)),
        pl.BlockSpec((D, D), lambda i: (0, 0)),
        pl.BlockSpec((D,), lambda i: (0,)),
        pl.BlockSpec((D,), lambda i: (0,)),
    ],
    out_specs=[
        pl.BlockSpec((NQ, BN, HQ), lambda i: (0, i, 0)),
        pl.BlockSpec((BN, H), lambda i: (i, 0)),
    ],
    out_shape=[
        jax.ShapeDtypeStruct((NQ, N, HQ), jnp.bfloat16),
        jax.ShapeDtypeStruct((N, H), _f32),
    ],
)


def _fin_body(o_ref, b_ref, g_ref, be_ref, out_ref):
    o = jnp.concatenate([o_ref[q] for q in range(NQ)], axis=1)
    h = jnp.maximum(o + b_ref[...][None, :], 0.0)
    out_ref[...] = _ln(h, g_ref, be_ref)


_fin = pl.pallas_call(
    _fin_body,
    grid=(GB,),
    in_specs=[
        pl.BlockSpec((NQ, BN, HQ), lambda i: (0, i, 0)),
        pl.BlockSpec((D,), lambda i: (0,)),
        pl.BlockSpec((D,), lambda i: (0,)),
        pl.BlockSpec((D,), lambda i: (0,)),
    ],
    out_specs=pl.BlockSpec((BN, D), lambda i: (i, 0)),
    out_shape=jax.ShapeDtypeStruct((N, D), _f32),
)


def _edge_body(g_ref, we1_ref, av1_ref, we2_ref, av2_ref,
               o1_ref, o2_ref, l1_ref, l2_ref, cs_ref):
    i = pl.program_id(0)
    g = g_ref[...]

    @pl.when(i == 0)
    def _():
        cs_ref[...] = jnp.zeros((1, 8 * DE), _f32)

    cs_ref[...] += jnp.sum(g, axis=0, keepdims=True)
    colsum = cs_ref[0]
    p = lax.broadcasted_iota(jnp.int32, (8 * DE, 8), 0)
    q = lax.broadcasted_iota(jnp.int32, (8 * DE, 8), 1)
    for we_ref, av_ref, o_ref, l_ref in (
        (we1_ref, av1_ref, o1_ref, l1_ref),
        (we2_ref, av2_ref, o2_ref, l2_ref),
    ):
        v = jnp.sum(we_ref[...] * av_ref[...][None, :], axis=1)
        vrow = jnp.concatenate([v] * 8)
        v8 = jnp.where((p // DE) == q, vrow[:, None], 0.0)
        o_ref[...] = jnp.dot(g, v8, preferred_element_type=_f32)
        l_ref[...] = jnp.full((1, 128), jnp.sum(colsum * vrow) / E, _f32)


_edge = pl.pallas_call(
    _edge_body,
    grid=(E // (8 * EB),),
    in_specs=[
        pl.BlockSpec((EB, 8 * DE), lambda i: (i, 0)),
        pl.BlockSpec((DE, D), lambda i: (0, 0)),
        pl.BlockSpec((D,), lambda i: (0,)),
        pl.BlockSpec((DE, D), lambda i: (0, 0)),
        pl.BlockSpec((D,), lambda i: (0,)),
    ],
    out_specs=[
        pl.BlockSpec((EB, 8), lambda i: (i, 0)),
        pl.BlockSpec((EB, 8), lambda i: (i, 0)),
        pl.BlockSpec((1, 128), lambda i: (0, 0)),
        pl.BlockSpec((1, 128), lambda i: (0, 0)),
    ],
    out_shape=[
        jax.ShapeDtypeStruct((E // 8, 8), _f32),
        jax.ShapeDtypeStruct((E // 8, 8), _f32),
        jax.ShapeDtypeStruct((1, 128), _f32),
        jax.ShapeDtypeStruct((1, 128), _f32),
    ],
    scratch_shapes=[pltpu.VMEM((1, 8 * DE), _f32)],
)


def _sc_body(s_hbm, d3_hbm, ae_hbm, as_hbm, ad_hbm, xl_hbm, out_hbm,
             va, vb, vden, vs, vdl, vex, rows, rowsb, redv,
             acc, sden, sred, gs0, gs1, ss0):
    cid = lax.axis_index("c")
    sid = lax.axis_index("s")
    base = sid * CH
    pltpu.sync_copy(s_hbm.at[pl.ds(base, CH)], vs)
    pltpu.sync_copy(d3_hbm.at[sid], vdl)
    pltpu.sync_copy(ae_hbm.at[pl.ds(base, CH)], vex)
    pltpu.sync_copy(as_hbm, va)
    pltpu.sync_copy(ad_hbm, vb)
    zero16 = jnp.zeros((L,), _f32)

    @pl.loop(0, NP // L)
    def _(i):
        vden[pl.ds(i * L, L)] = zero16

    @pl.when(sid == 0)
    def _():
        pltpu.sync_copy(vden, sden)

    def _zero_acc():
        @pl.loop(0, RB)
        def _(r):
            for k in range(HQ // L):
                rows[0, r, pl.ds(k * L, L)] = zero16

        @pl.when(sid < NS - 1)
        def _():
            for t in range(5):
                pltpu.sync_copy(rows.at[0, pl.ds(0, RB)],
                                acc.at[pl.ds(sid * 640 + t * RB, RB)])

        @pl.when(sid == NS - 1)
        def _():
            for t in range(5):
                pltpu.sync_copy(rows.at[0, pl.ds(0, 80)],
                                acc.at[pl.ds(9600 + t * 80, 80)])

    _zero_acc()

    cofs = 2 * cid * N
    minit = jnp.full((L,), -3.0e38, _f32)

    @pl.loop(0, NB, init_carry=minit)
    def p1(nb, mx):
        for k in range(RB // L):
            fl = pl.ds(nb * RB + k * L, L)
            sv = vs[fl]
            dv = vdl[nb, pl.ds(k * L, L)]
            a = plsc.load_gather(va, [sv]) + plsc.load_gather(vb, [dv]) + vex[fl]
            a = jnp.maximum(a, 0.2 * a)
            vex[fl] = a
            vs[fl] = sv + cofs
            mx = jnp.maximum(mx, a)
        return mx

    redv[0, :] = jnp.full((L,), jnp.max(p1), _f32)
    pltpu.sync_copy(redv.at[0], sred.at[sid])
    plsc.subcore_barrier()
    pltpu.sync_copy(sred, redv)
    gv = redv[0, :]
    for t in range(1, NS):
        gv = jnp.maximum(gv, redv[t, :])
    gm = jnp.max(gv)

    DW = 8

    def dwait(nb):
        pltpu.make_async_copy(vex.at[pl.ds(nb * RB, RB)], sden.at[vdl.at[nb]],
                              ss0).wait()

    @pl.loop(0, NB)
    def _(nb):
        for k in range(RB // L):
            fl = pl.ds(nb * RB + k * L, L)
            ev = jnp.exp(vex[fl] - gm)
            vex[fl] = ev
        pltpu.async_copy(vex.at[pl.ds(nb * RB, RB)], sden.at[vdl.at[nb]],
                         ss0, add=True)

        @pl.when(nb >= DW)
        def _():
            dwait(nb - DW)

    @pl.loop(NB - DW, NB)
    def _(nb):
        dwait(nb)

    plsc.subcore_barrier()
    pltpu.sync_copy(sden, vden)

    @pl.loop(0, NB)
    def _(nb):
        for k in range(RB // L):
            fl = pl.ds(nb * RB + k * L, L)
            dv = vdl[nb, pl.ds(k * L, L)]
            den = plsc.load_gather(vden, [dv])
            vex[fl] = vex[fl] / (den + 1e-16)

    gsems = (gs0, gs1)
    ssems = (ss0, ss0)

    def gstart(b, j):
        pltpu.async_copy(xl_hbm.at[vs.at[pl.ds(b * RB, RB)]], rowsb.at[j],
                         gsems[j])

    def gwait(b, j):
        pltpu.make_async_copy(xl_hbm.at[vs.at[pl.ds(b * RB, RB)]],
                              rowsb.at[j], gsems[j]).wait()

    def sstart(b, j):
        pltpu.async_copy(rows.at[j], acc.at[vdl.at[b]], ssems[j], add=True)

    def swait(b, j):
        pltpu.make_async_copy(rows.at[j], acc.at[vdl.at[b]], ssems[j]).wait()

    for pq in range(2):
        if pq == 1:
            plsc.subcore_barrier()

            @pl.loop(0, CH // L)
            def _(i):
                sl = pl.ds(i * L, L)
                vs[sl] = vs[sl] + N

            _zero_acc()
            plsc.subcore_barrier()

        gstart(0, 0)

        @pl.loop(0, NB, step=2)
        def _(b0):
            for j in range(2):
                b = b0 + j
                gwait(b, j)

                @pl.loop(0, RB // L)
                def _(rg):
                    wv = vex[pl.ds(b * RB + rg * L, L)]
                    for ri in range(L):
                        w = wv[ri]
                        r = rg * L + ri
                        for g in range(HQ // 32):
                            ab = rowsb[j, r, pl.ds(g * 32, 32)]
                            ua, ub = plsc.unpack(
                                ab, format=plsc.PackFormat.INTERLEAVED)
                            rows[j, r, pl.ds(g * 32, L)] = ua * w
                            rows[j, r, pl.ds(g * 32 + L, L)] = ub * w

                jp = 1 - j

                @pl.when(b >= 1)
                def _():
                    swait(b - 1, jp)

                @pl.when(b + 1 < NB)
                def _():
                    gstart(b + 1, jp)

                sstart(b, j)

        swait(NB - 1, (NB - 1) % 2)
        plsc.subcore_barrier()
        qd = 2 * cid + pq

        @pl.when(sid < NS - 1)
        def _():
            pltpu.sync_copy(acc.at[pl.ds(sid * 640, 640)],
                            out_hbm.at[qd, pl.ds(sid * 640, 640)])

        @pl.when(sid == NS - 1)
        def _():
            pltpu.sync_copy(acc.at[pl.ds(9600, 400)],
                            out_hbm.at[qd, pl.ds(9600, 400)])


import functools


@functools.lru_cache(maxsize=None)
def _build_sc():
  return pl.kernel(
    _sc_body,
    out_type=jax.ShapeDtypeStruct((NQ, N, HQ), _f32),
    mesh=plsc.VectorSubcoreMesh(core_axis_name="c", subcore_axis_name="s"),
    compiler_params=pltpu.CompilerParams(needs_layout_passes=False,
                                         use_tc_tiling_on_sc=False),
    scratch_types=[
        pltpu.VMEM((N,), _f32),
        pltpu.VMEM((N,), _f32),
        pltpu.VMEM((NP,), _f32),
        pltpu.VMEM((CH,), jnp.int32),
        pltpu.VMEM((NB, RB), jnp.int32),
        pltpu.VMEM((CH,), _f32),
        pltpu.VMEM((2, RB, HQ), _f32),
        pltpu.VMEM((2, RB, HQ), jnp.bfloat16),
        pltpu.VMEM((NS, L), _f32),
        pltpu.VMEM_SHARED((N, HQ), _f32),
        pltpu.VMEM_SHARED((NP,), _f32),
        pltpu.VMEM_SHARED((NS, L), _f32),
        pltpu.SemaphoreType.DMA,
        pltpu.SemaphoreType.DMA,
        pltpu.SemaphoreType.DMA,
    ],
  )


def kernel(x, edge_index, edge_attr, W1, a_src1, a_dst1, We1, a_edge1, b1, g1,
           be1, W2, a_src2, a_dst2, We2, a_edge2, b2, g2, be2):
    src = edge_index[0].astype(jnp.int32)
    dst = edge_index[1].astype(jnp.int32)
    gea = edge_attr.reshape(E // 8, 8 * DE)
    ae1g, ae2g, l1, l2 = _edge(gea, We1, a_edge1, We2, a_edge2)
    ar = jnp.arange(N, dtype=jnp.int32)
    zpad = jnp.zeros((PAD,), jnp.int32)
    s_full = jnp.concatenate([src, ar, zpad])
    d_full = jnp.concatenate([dst, ar, zpad])
    d3 = d_full.reshape(NS, NB, RB)
    neg = jnp.full((PAD,), -1e30, _f32)
    ae_f1 = jnp.concatenate([ae1g.reshape(E), jnp.full((N,), l1[0, 0]), neg])
    ae_f2 = jnp.concatenate([ae2g.reshape(E), jnp.full((N,), l2[0, 0]), neg])

    m = jnp.arange(D)
    perm = (m // 32) * 32 + (m % 32) // 2 + 16 * (m % 2)
    W1p, as1p, ad1p = W1[:, perm], a_src1[perm], a_dst1[perm]
    W2p, as2p, ad2p = W2[:, perm], a_src2[perm], a_dst2[perm]

    xq1, xa1 = _node(x, W1p, as1p, ad1p)
    _sc = _build_sc()
    oc1 = _sc(s_full, d3, ae_f1, xa1[:, 0], xa1[:, 1], xq1.reshape(NQ * N, HQ))

    xq2, xa2 = _mid(oc1, b1, g1, be1, W2p, as2p, ad2p)
    oc2 = _sc(s_full, d3, ae_f2, xa2[:, 0], xa2[:, 1], xq2.reshape(NQ * N, HQ))

    return _fin(oc2, b2, g2, be2)


# speedup vs baseline: 1.4808x; 1.4808x over previous
import jax
import jax.numpy as jnp
from jax import lax
from jax.experimental import pallas as pl
from jax.experimental.pallas import tpu as pltpu
from jax.experimental.pallas import tpu_sc as plsc

N = 10000
E = 160000
D = 256
DE = 16
H = 128
NC = 2
NS = 16
L = 16
RB = 128
CH = 10752
EP = CH * NS
NB = CH // RB
PAD = EP - E - N
NROW = N // NS
HQ = 64
NQ = 4
NP = 10240
NRD = NP // NS
GB = 10
BN = N // GB
EB = 2000

_f32 = jnp.float32


def _wg(w_ref, avs_ref, avd_ref):
    w = w_ref[...]
    cs = jnp.sum(w * avs_ref[...][None, :], axis=1)
    cd = jnp.sum(w * avd_ref[...][None, :], axis=1)
    return jnp.concatenate(
        [w, cs[:, None], cd[:, None], jnp.zeros((D, H - 2), _f32)], axis=1)


def _split_out(xl, oq_ref, oa_ref):
    for q in range(NQ):
        oq_ref[q] = xl[:, q * HQ:(q + 1) * HQ]
    oa_ref[...] = xl[:, D:]


def _node_body(x_ref, w_ref, avs_ref, avd_ref, oq_ref, oa_ref):
    xl = jnp.dot(x_ref[...], _wg(w_ref, avs_ref, avd_ref),
                 preferred_element_type=_f32)
    _split_out(xl, oq_ref, oa_ref)


_node = pl.pallas_call(
    _node_body,
    grid=(GB,),
    in_specs=[
        pl.BlockSpec((BN, D), lambda i: (i, 0)),
        pl.BlockSpec((D, D), lambda i: (0, 0)),
        pl.BlockSpec((D,), lambda i: (0,)),
        pl.BlockSpec((D,), lambda i: (0,)),
    ],
    out_specs=[
        pl.BlockSpec((NQ, BN, HQ), lambda i: (0, i, 0)),
        pl.BlockSpec((BN, H), lambda i: (i, 0)),
    ],
    out_shape=[
        jax.ShapeDtypeStruct((NQ, N, HQ), _f32),
        jax.ShapeDtypeStruct((N, H), _f32),
    ],
)


def _ln(h, g_ref, be_ref):
    m = jnp.mean(h, axis=1, keepdims=True)
    v = jnp.mean((h - m) ** 2, axis=1, keepdims=True)
    return (h - m) / jnp.sqrt(v + 1e-5) * g_ref[...][None, :] + be_ref[...][None, :]


def _mid_body(o_ref, b_ref, g_ref, be_ref, w_ref, avs_ref, avd_ref,
              oq_ref, oa_ref):
    o = jnp.concatenate([o_ref[q] for q in range(NQ)], axis=1)
    h = jnp.maximum(o + b_ref[...][None, :], 0.0)
    h = _ln(h, g_ref, be_ref)
    xl = jnp.dot(h, _wg(w_ref, avs_ref, avd_ref), preferred_element_type=_f32)
    _split_out(xl, oq_ref, oa_ref)


_mid = pl.pallas_call(
    _mid_body,
    grid=(GB,),
    in_specs=[
        pl.BlockSpec((NQ, BN, HQ), lambda i: (0, i, 0)),
        pl.BlockSpec((D,), lambda i: (0,)),
        pl.BlockSpec((D,), lambda i: (0,)),
        pl.BlockSpec((D,), lambda i: (0,)),
        pl.BlockSpec((D, D), lambda i: (0, 0)),
        pl.BlockSpec((D,), lambda i: (0,)),
        pl.BlockSpec((D,), lambda i: (0,)),
    ],
    out_specs=[
        pl.BlockSpec((NQ, BN, HQ), lambda i: (0, i, 0)),
        pl.BlockSpec((BN, H), lambda i: (i, 0)),
    ],
    out_shape=[
        jax.ShapeDtypeStruct((NQ, N, HQ), _f32),
        jax.ShapeDtypeStruct((N, H), _f32),
    ],
)


def _fin_body(o_ref, b_ref, g_ref, be_ref, out_ref):
    o = jnp.concatenate([o_ref[q] for q in range(NQ)], axis=1)
    h = jnp.maximum(o + b_ref[...][None, :], 0.0)
    out_ref[...] = _ln(h, g_ref, be_ref)


_fin = pl.pallas_call(
    _fin_body,
    grid=(GB,),
    in_specs=[
        pl.BlockSpec((NQ, BN, HQ), lambda i: (0, i, 0)),
        pl.BlockSpec((D,), lambda i: (0,)),
        pl.BlockSpec((D,), lambda i: (0,)),
        pl.BlockSpec((D,), lambda i: (0,)),
    ],
    out_specs=pl.BlockSpec((BN, D), lambda i: (i, 0)),
    out_shape=jax.ShapeDtypeStruct((N, D), _f32),
)


def _edge_body(g_ref, we1_ref, av1_ref, we2_ref, av2_ref,
               o1_ref, o2_ref, l1_ref, l2_ref, cs_ref):
    i = pl.program_id(0)
    g = g_ref[...]

    @pl.when(i == 0)
    def _():
        cs_ref[...] = jnp.zeros((1, 8 * DE), _f32)

    cs_ref[...] += jnp.sum(g, axis=0, keepdims=True)
    colsum = cs_ref[0]
    p = lax.broadcasted_iota(jnp.int32, (8 * DE, 8), 0)
    q = lax.broadcasted_iota(jnp.int32, (8 * DE, 8), 1)
    for we_ref, av_ref, o_ref, l_ref in (
        (we1_ref, av1_ref, o1_ref, l1_ref),
        (we2_ref, av2_ref, o2_ref, l2_ref),
    ):
        v = jnp.sum(we_ref[...] * av_ref[...][None, :], axis=1)
        vrow = jnp.concatenate([v] * 8)
        v8 = jnp.where((p // DE) == q, vrow[:, None], 0.0)
        o_ref[...] = jnp.dot(g, v8, preferred_element_type=_f32)
        l_ref[...] = jnp.full((1, 128), jnp.sum(colsum * vrow) / E, _f32)


_edge = pl.pallas_call(
    _edge_body,
    grid=(E // (8 * EB),),
    in_specs=[
        pl.BlockSpec((EB, 8 * DE), lambda i: (i, 0)),
        pl.BlockSpec((DE, D), lambda i: (0, 0)),
        pl.BlockSpec((D,), lambda i: (0,)),
        pl.BlockSpec((DE, D), lambda i: (0, 0)),
        pl.BlockSpec((D,), lambda i: (0,)),
    ],
    out_specs=[
        pl.BlockSpec((EB, 8), lambda i: (i, 0)),
        pl.BlockSpec((EB, 8), lambda i: (i, 0)),
        pl.BlockSpec((1, 128), lambda i: (0, 0)),
        pl.BlockSpec((1, 128), lambda i: (0, 0)),
    ],
    out_shape=[
        jax.ShapeDtypeStruct((E // 8, 8), _f32),
        jax.ShapeDtypeStruct((E // 8, 8), _f32),
        jax.ShapeDtypeStruct((1, 128), _f32),
        jax.ShapeDtypeStruct((1, 128), _f32),
    ],
    scratch_shapes=[pltpu.VMEM((1, 8 * DE), _f32)],
)


def _sc_body(s_hbm, d3_hbm, ae_hbm, as_hbm, ad_hbm, xl_hbm, out_hbm,
             va, vb, vden, vs, vdl, vex, rows, redv,
             acc, sden, sred, gs0, gs1, gs2, ss0):
    cid = lax.axis_index("c")
    sid = lax.axis_index("s")
    base = sid * CH
    pltpu.sync_copy(s_hbm.at[pl.ds(base, CH)], vs)
    pltpu.sync_copy(d3_hbm.at[sid], vdl)
    pltpu.sync_copy(ae_hbm.at[pl.ds(base, CH)], vex)
    pltpu.sync_copy(as_hbm, va)
    pltpu.sync_copy(ad_hbm, vb)
    zero16 = jnp.zeros((L,), _f32)

    @pl.loop(0, NP // L)
    def _(i):
        vden[pl.ds(i * L, L)] = zero16

    @pl.when(sid == 0)
    def _():
        pltpu.sync_copy(vden, sden)

    def _zero_acc():
        @pl.loop(0, RB)
        def _(r):
            for k in range(HQ // L):
                rows[0, r, pl.ds(k * L, L)] = zero16

        @pl.when(sid < NS - 1)
        def _():
            for t in range(5):
                pltpu.sync_copy(rows.at[0, pl.ds(0, RB)],
                                acc.at[pl.ds(sid * 640 + t * RB, RB)])

        @pl.when(sid == NS - 1)
        def _():
            for t in range(5):
                pltpu.sync_copy(rows.at[0, pl.ds(0, 80)],
                                acc.at[pl.ds(9600 + t * 80, 80)])

    _zero_acc()

    cofs = 2 * cid * N
    minit = jnp.full((L,), -3.0e38, _f32)

    @pl.loop(0, NB, init_carry=minit)
    def p1(nb, mx):
        for k in range(RB // L):
            fl = pl.ds(nb * RB + k * L, L)
            sv = vs[fl]
            dv = vdl[nb, pl.ds(k * L, L)]
            a = plsc.load_gather(va, [sv]) + plsc.load_gather(vb, [dv]) + vex[fl]
            a = jnp.maximum(a, 0.2 * a)
            vex[fl] = a
            vs[fl] = sv + cofs
            mx = jnp.maximum(mx, a)
        return mx

    redv[0, :] = jnp.full((L,), jnp.max(p1), _f32)
    pltpu.sync_copy(redv.at[0], sred.at[sid])
    plsc.subcore_barrier()
    pltpu.sync_copy(sred, redv)
    gv = redv[0, :]
    for t in range(1, NS):
        gv = jnp.maximum(gv, redv[t, :])
    gm = jnp.max(gv)

    DW = 8

    def dwait(nb):
        pltpu.make_async_copy(vex.at[pl.ds(nb * RB, RB)], sden.at[vdl.at[nb]],
                              ss0).wait()

    @pl.loop(0, NB)
    def _(nb):
        for k in range(RB // L):
            fl = pl.ds(nb * RB + k * L, L)
            ev = jnp.exp(vex[fl] - gm)
            vex[fl] = ev
        pltpu.async_copy(vex.at[pl.ds(nb * RB, RB)], sden.at[vdl.at[nb]],
                         ss0, add=True)

        @pl.when(nb >= DW)
        def _():
            dwait(nb - DW)

    @pl.loop(NB - DW, NB)
    def _(nb):
        dwait(nb)

    plsc.subcore_barrier()
    pltpu.sync_copy(sden, vden)

    @pl.loop(0, NB)
    def _(nb):
        for k in range(RB // L):
            fl = pl.ds(nb * RB + k * L, L)
            dv = vdl[nb, pl.ds(k * L, L)]
            den = plsc.load_gather(vden, [dv])
            vex[fl] = vex[fl] / (den + 1e-16)

    gsems = (gs0, gs1, gs2)
    ssems = (ss0, ss0, ss0)

    def gstart(b, j):
        pltpu.async_copy(xl_hbm.at[vs.at[pl.ds(b * RB, RB)]], rows.at[j],
                         gsems[j])

    def gwait(b, j):
        pltpu.make_async_copy(xl_hbm.at[vs.at[pl.ds(b * RB, RB)]],
                              rows.at[j], gsems[j]).wait()

    def sstart(b, j):
        pltpu.async_copy(rows.at[j], acc.at[vdl.at[b]], ssems[j], add=True)

    def swait(b, j):
        pltpu.make_async_copy(rows.at[j], acc.at[vdl.at[b]], ssems[j]).wait()

    for pq in range(2):
        if pq == 1:
            plsc.subcore_barrier()

            @pl.loop(0, CH // L)
            def _(i):
                sl = pl.ds(i * L, L)
                vs[sl] = vs[sl] + N

            _zero_acc()
            plsc.subcore_barrier()

        gstart(0, 0)
        gstart(1, 1)

        @pl.loop(0, NB, step=3)
        def _(b0):
            for j in range(3):
                b = b0 + j
                gwait(b, j)

                @pl.loop(0, RB // L)
                def _(rg):
                    wv = vex[pl.ds(b * RB + rg * L, L)]
                    for ri in range(L):
                        w = wv[ri]
                        r = rg * L + ri
                        for k in range(HQ // L):
                            sl = pl.ds(k * L, L)
                            rows[j, r, sl] = rows[j, r, sl] * w

                jp = (j + 2) % 3

                @pl.when(b >= 1)
                def _():
                    swait(b - 1, jp)

                @pl.when(b + 2 < NB)
                def _():
                    gstart(b + 2, jp)

                sstart(b, j)

        swait(NB - 1, (NB - 1) % 3)
        plsc.subcore_barrier()
        qd = 2 * cid + pq

        @pl.when(sid < NS - 1)
        def _():
            pltpu.sync_copy(acc.at[pl.ds(sid * 640, 640)],
                            out_hbm.at[qd, pl.ds(sid * 640, 640)])

        @pl.when(sid == NS - 1)
        def _():
            pltpu.sync_copy(acc.at[pl.ds(9600, 400)],
                            out_hbm.at[qd, pl.ds(9600, 400)])


import functools


@functools.lru_cache(maxsize=None)
def _build_sc():
  return pl.kernel(
    _sc_body,
    out_type=jax.ShapeDtypeStruct((NQ, N, HQ), _f32),
    mesh=plsc.VectorSubcoreMesh(core_axis_name="c", subcore_axis_name="s"),
    compiler_params=pltpu.CompilerParams(needs_layout_passes=False,
                                         use_tc_tiling_on_sc=False),
    scratch_types=[
        pltpu.VMEM((N,), _f32),
        pltpu.VMEM((N,), _f32),
        pltpu.VMEM((NP,), _f32),
        pltpu.VMEM((CH,), jnp.int32),
        pltpu.VMEM((NB, RB), jnp.int32),
        pltpu.VMEM((CH,), _f32),
        pltpu.VMEM((3, RB, HQ), _f32),
        pltpu.VMEM((NS, L), _f32),
        pltpu.VMEM_SHARED((N, HQ), _f32),
        pltpu.VMEM_SHARED((NP,), _f32),
        pltpu.VMEM_SHARED((NS, L), _f32),
        pltpu.SemaphoreType.DMA,
        pltpu.SemaphoreType.DMA,
        pltpu.SemaphoreType.DMA,
        pltpu.SemaphoreType.DMA,
    ],
  )


def kernel(x, edge_index, edge_attr, W1, a_src1, a_dst1, We1, a_edge1, b1, g1,
           be1, W2, a_src2, a_dst2, We2, a_edge2, b2, g2, be2):
    src = edge_index[0].astype(jnp.int32)
    dst = edge_index[1].astype(jnp.int32)
    gea = edge_attr.reshape(E // 8, 8 * DE)
    ae1g, ae2g, l1, l2 = _edge(gea, We1, a_edge1, We2, a_edge2)
    ar = jnp.arange(N, dtype=jnp.int32)
    zpad = jnp.zeros((PAD,), jnp.int32)
    s_full = jnp.concatenate([src, ar, zpad])
    d_full = jnp.concatenate([dst, ar, zpad])
    d3 = d_full.reshape(NS, NB, RB)
    neg = jnp.full((PAD,), -1e30, _f32)
    ae_f1 = jnp.concatenate([ae1g.reshape(E), jnp.full((N,), l1[0, 0]), neg])
    ae_f2 = jnp.concatenate([ae2g.reshape(E), jnp.full((N,), l2[0, 0]), neg])

    xq1, xa1 = _node(x, W1, a_src1, a_dst1)
    _sc = _build_sc()
    oc1 = _sc(s_full, d3, ae_f1, xa1[:, 0], xa1[:, 1], xq1.reshape(NQ * N, HQ))

    xq2, xa2 = _mid(oc1, b1, g1, be1, W2, a_src2, a_dst2)
    oc2 = _sc(s_full, d3, ae_f2, xa2[:, 0], xa2[:, 1], xq2.reshape(NQ * N, HQ))

    return _fin(oc2, b2, g2, be2)
